# Initial kernel scaffold; baseline (speedup 1.0000x reference)
import jax
import jax.numpy as jnp
from jax.experimental import pallas as pl


def kernel(boxes, scores):
    raise NotImplementedError("write your pallas kernel here")


# speedup vs baseline: 20.4690x; 20.4690x over previous
import jax
import jax.numpy as jnp
from jax.experimental import pallas as pl
from jax.experimental.pallas import tpu as pltpu

_IOU_THRESHOLD = 0.8
_K_SELECT = 1024
_N = 20000
_ROWS = 160
_COLS = 128
_N_PAD = _ROWS * _COLS
_NEG_INF = float("-inf")


def _nms_body(x1_ref, y1_ref, x2_ref, y2_ref, sc_ref, out_ref, valid_ref):
    x1 = x1_ref[...]
    y1 = y1_ref[...]
    x2 = x2_ref[...]
    y2 = y2_ref[...]
    sc = sc_ref[...]
    areas = (x2 - x1) * (y2 - y1)

    ridx = jax.lax.broadcasted_iota(jnp.int32, (_ROWS, _COLS), 0)
    cidx = jax.lax.broadcasted_iota(jnp.int32, (_ROWS, _COLS), 1)
    gidx = ridx * _COLS + cidx
    lane = jax.lax.broadcasted_iota(jnp.int32, (1, _COLS), 1)

    def body(i, _):
        valid = valid_ref[...] > 0.0
        masked = jnp.where(valid, sc, _NEG_INF)
        m = jnp.max(masked)
        idx = jnp.min(jnp.where(masked == m, gidx, _N_PAD))
        sel = gidx == idx
        x1i = jnp.max(jnp.where(sel, x1, _NEG_INF))
        y1i = jnp.max(jnp.where(sel, y1, _NEG_INF))
        x2i = jnp.max(jnp.where(sel, x2, _NEG_INF))
        y2i = jnp.max(jnp.where(sel, y2, _NEG_INF))
        si = jnp.max(jnp.where(sel, sc, _NEG_INF))
        area_i = (x2i - x1i) * (y2i - y1i)

        row = (
            jnp.where(lane == 0, x1i, 0.0)
            + jnp.where(lane == 1, y1i, 0.0)
            + jnp.where(lane == 2, x2i, 0.0)
            + jnp.where(lane == 3, y2i, 0.0)
            + jnp.where(lane == 4, si, 0.0)
        )
        out_ref[pl.ds(i, 1), :] = row

        xx1 = jnp.maximum(x1i, x1)
        yy1 = jnp.maximum(y1i, y1)
        xx2 = jnp.minimum(x2i, x2)
        yy2 = jnp.minimum(y2i, y2)
        inter = jnp.maximum(xx2 - xx1, 0.0) * jnp.maximum(yy2 - yy1, 0.0)
        iou = inter / (area_i + areas - inter + 1e-8)
        keep_mask = jnp.logical_not(iou > _IOU_THRESHOLD)
        new_valid = valid & keep_mask & jnp.logical_not(sel)
        valid_ref[...] = jnp.where(new_valid, 1.0, 0.0)
        return 0

    valid_ref[...] = jnp.where(gidx < _N, 1.0, 0.0)
    jax.lax.fori_loop(0, _K_SELECT, body, 0)


def kernel(boxes, scores):
    pad = _N_PAD - _N
    x1 = jnp.pad(boxes[:, 0], (0, pad)).reshape(_ROWS, _COLS)
    y1 = jnp.pad(boxes[:, 1], (0, pad)).reshape(_ROWS, _COLS)
    x2 = jnp.pad(boxes[:, 2], (0, pad)).reshape(_ROWS, _COLS)
    y2 = jnp.pad(boxes[:, 3], (0, pad)).reshape(_ROWS, _COLS)
    sc = jnp.pad(scores, (0, pad)).reshape(_ROWS, _COLS)

    out = pl.pallas_call(
        _nms_body,
        out_shape=jax.ShapeDtypeStruct((_K_SELECT, _COLS), jnp.float32),
        scratch_shapes=[pltpu.VMEM((_ROWS, _COLS), jnp.float32)],
    )(x1, y1, x2, y2, sc)
    return out[:, :5]
